# Initial kernel scaffold; baseline (speedup 1.0000x reference)
import jax
import jax.numpy as jnp
from jax.experimental import pallas as pl


def kernel(x, tok_emb, pos_emb, ln_attn_g, ln_attn_b, wq, bq, wk, bk, wv, bv, wo, bo, ln_ff_g, ln_ff_b, moe_ln_g, moe_ln_b, wg, w1, b1, w2, b2, ln_head_g, ln_head_b):
    raise NotImplementedError("write your pallas kernel here")


# speedup vs baseline: 1.0563x; 1.0563x over previous
import functools

import jax
import jax.numpy as jnp
from jax import lax
from jax.experimental import pallas as pl
from jax.experimental.pallas import tpu as pltpu
from jax.experimental.pallas import tpu_sc as plsc

_B, _S, _D, _H, _DH, _L, _E, _K, _V, _DFF = 1, 2048, 768, 12, 64, 2, 8, 2, 32000, 3072
_T = _B * _S
_CAP = 640
_ECAP = _E * _CAP

_NC, _NS = 2, 16
_NW = _NC * _NS

def _sc_mesh():
    return dict(
        mesh=plsc.VectorSubcoreMesh(
            core_axis_name="c", subcore_axis_name="s", num_cores=_NC),
        compiler_params=pltpu.CompilerParams(needs_layout_passes=False))


def _worker_id():
    return lax.axis_index("s") * _NC + lax.axis_index("c")


def _sc_gather(table, idx, chunk):
    n, d = idx.shape[0], table.shape[1]
    per_w = n // _NW
    assert per_w % chunk == 0 and chunk % 8 == 0

    @functools.partial(
        pl.kernel,
        out_type=jax.ShapeDtypeStruct((n, d), jnp.float32),
        scratch_types=[
            pltpu.VMEM((chunk,), jnp.int32),
            pltpu.VMEM((chunk, d), jnp.float32),
            pltpu.SemaphoreType.DMA,
        ],
        **_sc_mesh(),
    )
    def k(table_hbm, idx_hbm, out_hbm, idx_v, rows_v, sem):
        base0 = _worker_id() * per_w
        for c in range(per_w // chunk):
            base = base0 + c * chunk
            pltpu.sync_copy(idx_hbm.at[pl.ds(base, chunk)], idx_v)
            pltpu.async_copy(table_hbm.at[idx_v], rows_v, sem).wait()
            pltpu.sync_copy(rows_v, out_hbm.at[pl.ds(base, chunk)])

    return k(table, idx)


def _sc_dispatch(xt, slot_a, slot_b, valid_a, valid_b, zeros_ecap):
    per_w = _ECAP // _NW
    chunk = 80

    @functools.partial(
        pl.kernel,
        out_type=jax.ShapeDtypeStruct((_ECAP, _D), jnp.float32),
        scratch_types=[
            pltpu.VMEM((_ECAP,), jnp.int32),
            pltpu.VMEM((_T,), jnp.int32),
            pltpu.VMEM((_T,), jnp.int32),
            pltpu.VMEM((_T,), jnp.int32),
            pltpu.VMEM((_T,), jnp.int32),
            pltpu.VMEM((chunk, _D), jnp.float32),
            pltpu.SemaphoreType.DMA,
        ],
        **_sc_mesh(),
    )
    def k(xt_hbm, sa_hbm, sb_hbm, va_hbm, vb_hbm, z_hbm, out_hbm,
          inv_v, sa_v, sb_v, va_v, vb_v, rows_v, sem):
        pltpu.sync_copy(z_hbm, inv_v)
        pltpu.sync_copy(sa_hbm, sa_v)
        pltpu.sync_copy(sb_hbm, sb_v)
        pltpu.sync_copy(va_hbm, va_v)
        pltpu.sync_copy(vb_hbm, vb_v)

        def body(i, _):
            toks = lax.iota(jnp.int32, 16) + i * 16
            plsc.store_scatter(inv_v, [sa_v[pl.ds(i * 16, 16)]], toks,
                               mask=va_v[pl.ds(i * 16, 16)] > 0)
            plsc.store_scatter(inv_v, [sb_v[pl.ds(i * 16, 16)]], toks,
                               mask=vb_v[pl.ds(i * 16, 16)] > 0)
            return 0

        lax.fori_loop(0, _T // 16, body, 0)

        base0 = _worker_id() * per_w
        for c in range(per_w // chunk):
            base = base0 + c * chunk
            pltpu.async_copy(xt_hbm.at[inv_v.at[pl.ds(base, chunk)]], rows_v,
                             sem).wait()
            pltpu.sync_copy(rows_v, out_hbm.at[pl.ds(base, chunk)])

    return k(xt, slot_a, slot_b, valid_a, valid_b, zeros_ecap)


def _ln(h, g, b):
    m = jnp.mean(h, axis=-1, keepdims=True)
    v = jnp.mean((h - m) ** 2, axis=-1, keepdims=True)
    return (h - m) / jnp.sqrt(v + 1e-5) * g + b


def _make_lnqkv(with_pos):
    nb = 8
    blk = _T // nb

    def body(*refs):
        if with_pos:
            (h_ref, pos_ref, g_ref, b_ref, wq_ref, bq_ref, wk_ref, bk_ref,
             wv_ref, bv_ref, q_ref, k_ref, v_ref, hres_ref) = refs
            hh = h_ref[...] + pos_ref[...]
            hres_ref[...] = hh
        else:
            (h_ref, g_ref, b_ref, wq_ref, bq_ref, wk_ref, bk_ref,
             wv_ref, bv_ref, q_ref, k_ref, v_ref) = refs
            hh = h_ref[...]
        a = _ln(hh, g_ref[...], b_ref[...])
        q_ref[...] = jnp.dot(a, wq_ref[...], preferred_element_type=jnp.float32) + bq_ref[...]
        k_ref[...] = jnp.dot(a, wk_ref[...], preferred_element_type=jnp.float32) + bk_ref[...]
        v_ref[...] = jnp.dot(a, wv_ref[...], preferred_element_type=jnp.float32) + bv_ref[...]

    row_spec = pl.BlockSpec((blk, _D), lambda i: (i, 0))
    full_spec = pl.BlockSpec((_D, _D), lambda i: (0, 0))
    vec_spec = pl.BlockSpec((1, _D), lambda i: (0, 0))
    n_out = 4 if with_pos else 3
    in_specs = [row_spec] * (2 if with_pos else 1) + [vec_spec, vec_spec] + \
        [full_spec, vec_spec] * 3
    out_specs = [row_spec] * n_out
    out_shape = [jax.ShapeDtypeStruct((_T, _D), jnp.float32)] * n_out
    return pl.pallas_call(body, grid=(nb,), in_specs=in_specs,
                          out_specs=out_specs, out_shape=out_shape)


def _attn(q, k, v, pad_row):
    qb = 512
    nq = _T // qb

    def body(q_ref, k_ref, v_ref, pad_ref, o_ref):
        qi = pl.program_id(1)
        qv = q_ref[...].reshape(qb, _DH)
        kv = k_ref[...].reshape(_T, _DH)
        s = lax.dot_general(qv, kv, (((1,), (1,)), ((), ())),
                            preferred_element_type=jnp.float32) / 8.0
        rows = qi * qb + lax.broadcasted_iota(jnp.int32, (qb, _T), 0)
        cols = lax.broadcasted_iota(jnp.int32, (qb, _T), 1)
        masked = (cols > rows) | (pad_ref[...] > 0.0)
        s = jnp.where(masked, -1e9, s)
        m = jnp.max(s, axis=1, keepdims=True)
        p = jnp.exp(s - m)
        l = jnp.sum(p, axis=1, keepdims=True)
        o = lax.dot_general(p, v_ref[...].reshape(_T, _DH),
                            (((1,), (0,)), ((), ())),
                            preferred_element_type=jnp.float32) / l
        o_ref[...] = o.reshape(1, qb, _DH)

    return pl.pallas_call(
        body,
        grid=(_H, nq),
        in_specs=[
            pl.BlockSpec((1, qb, _DH), lambda h, i: (h, i, 0)),
            pl.BlockSpec((1, _T, _DH), lambda h, i: (h, 0, 0)),
            pl.BlockSpec((1, _T, _DH), lambda h, i: (h, 0, 0)),
            pl.BlockSpec((1, _T), lambda h, i: (0, 0)),
        ],
        out_specs=pl.BlockSpec((1, qb, _DH), lambda h, i: (h, i, 0)),
        out_shape=jax.ShapeDtypeStruct((_H, _T, _DH), jnp.float32),
    )(q, k, v, pad_row)


def _oproj(o, h, wo, bo, g, b):
    nb = 8
    blk = _T // nb

    def body(o_ref, h_ref, wo_ref, bo_ref, g_ref, b_ref, h2_ref, t_ref):
        h2 = h_ref[...] + jnp.dot(o_ref[...], wo_ref[...],
                                  preferred_element_type=jnp.float32) + bo_ref[...]
        h2_ref[...] = h2
        t_ref[...] = _ln(h2, g_ref[...], b_ref[...])

    row_spec = pl.BlockSpec((blk, _D), lambda i: (i, 0))
    return pl.pallas_call(
        body,
        grid=(nb,),
        in_specs=[row_spec, row_spec,
                  pl.BlockSpec((_D, _D), lambda i: (0, 0)),
                  pl.BlockSpec((1, _D), lambda i: (0, 0)),
                  pl.BlockSpec((1, _D), lambda i: (0, 0)),
                  pl.BlockSpec((1, _D), lambda i: (0, 0))],
        out_specs=[row_spec, row_spec],
        out_shape=[jax.ShapeDtypeStruct((_T, _D), jnp.float32)] * 2,
    )(o, h, wo, bo, g, b)


def _router(xt, wg):
    cb = 256
    nb = _T // cb

    def body(xt_ref, wg_ref, sa_ref, sb_ref, ca_ref, cb_ref, va_ref, vb_ref):
        gl = jnp.dot(xt_ref[...], wg_ref[...], preferred_element_type=jnp.float32)
        iota_e = lax.broadcasted_iota(jnp.int32, (_T, _E), 1)
        m1 = jnp.max(gl, axis=1, keepdims=True)
        i1 = jnp.min(jnp.where(gl == m1, iota_e, _E), axis=1, keepdims=True)
        gl2 = jnp.where(iota_e == i1, -1e30, gl)
        m2 = jnp.max(gl2, axis=1, keepdims=True)
        i2 = jnp.min(jnp.where(gl2 == m2, iota_e, _E), axis=1, keepdims=True)
        e = jnp.exp(m2 - m1)
        g1 = 1.0 / (1.0 + e)
        g2 = e / (1.0 + e)
        oha = (iota_e == i1).astype(jnp.float32)
        ohb = (iota_e == i2).astype(jnp.float32)
        oh = oha + ohb
        tri = (lax.broadcasted_iota(jnp.int32, (cb, cb), 1)
               < lax.broadcasted_iota(jnp.int32, (cb, cb), 0)).astype(jnp.float32)
        run = jnp.zeros((1, _E), jnp.float32)
        cs = []
        for bi in range(nb):
            blk = oh[bi * cb:(bi + 1) * cb]
            cs.append(jnp.dot(tri, blk, preferred_element_type=jnp.float32) + run)
            run = run + jnp.sum(blk, axis=0, keepdims=True)
        c = jnp.concatenate(cs, axis=0)
        pie_a = jnp.sum(c * oha, axis=1, keepdims=True).astype(jnp.int32)
        pie_b = jnp.sum(c * ohb, axis=1, keepdims=True).astype(jnp.int32)
        va = (pie_a < _CAP).astype(jnp.int32)
        vb = (pie_b < _CAP).astype(jnp.int32)
        sa_ref[...] = i1 * _CAP + jnp.minimum(pie_a, _CAP - 1)
        sb_ref[...] = i2 * _CAP + jnp.minimum(pie_b, _CAP - 1)
        ca_ref[...] = g1 * va.astype(jnp.float32)
        cb_ref[...] = g2 * vb.astype(jnp.float32)
        va_ref[...] = va
        vb_ref[...] = vb

    col_i = jax.ShapeDtypeStruct((_T, 1), jnp.int32)
    col_f = jax.ShapeDtypeStruct((_T, 1), jnp.float32)
    return pl.pallas_call(
        body,
        in_specs=[pl.BlockSpec((_T, _D), lambda: (0, 0)),
                  pl.BlockSpec((_D, _E), lambda: (0, 0))],
        out_specs=[pl.BlockSpec((_T, 1), lambda: (0, 0))] * 6,
        out_shape=[col_i, col_i, col_f, col_f, col_i, col_i],
    )(xt, wg)


def _expert_ffn(eb, w1, b1, w2, b2):
    nj = _DFF // _D

    def body(eb_ref, w1_ref, b1_ref, w2_ref, b2_ref, out_ref):
        j = pl.program_id(1)
        ebv = eb_ref[...].reshape(_CAP, _D)
        hid = jnp.maximum(
            jnp.dot(ebv, w1_ref[...].reshape(_D, _D),
                    preferred_element_type=jnp.float32)
            + b1_ref[...].reshape(1, _D), 0.0)
        part = jnp.dot(hid, w2_ref[...].reshape(_D, _D),
                       preferred_element_type=jnp.float32)

        @pl.when(j == 0)
        def _():
            out_ref[...] = (part + b2_ref[...].reshape(1, _D))[None]

        @pl.when(j > 0)
        def _():
            out_ref[...] = out_ref[...] + part[None]

    return pl.pallas_call(
        body,
        grid=(_E, nj),
        in_specs=[
            pl.BlockSpec((1, _CAP, _D), lambda e, j: (e, 0, 0)),
            pl.BlockSpec((1, _D, _D), lambda e, j: (e, 0, j)),
            pl.BlockSpec((1, 1, _D), lambda e, j: (e, 0, j)),
            pl.BlockSpec((1, _D, _D), lambda e, j: (e, j, 0)),
            pl.BlockSpec((1, 1, _D), lambda e, j: (e, 0, 0)),
        ],
        out_specs=pl.BlockSpec((1, _CAP, _D), lambda e, j: (e, 0, 0)),
        out_shape=jax.ShapeDtypeStruct((_E, _CAP, _D), jnp.float32),
    )(eb, w1, b1.reshape(_E, 1, _DFF), w2, b2.reshape(_E, 1, _D))


def _combine(ga, gb, ca, cb, t, h2, g, b):
    nb = 8
    blk = _T // nb

    def body(ga_ref, gb_ref, ca_ref, cb_ref, t_ref, h2_ref, g_ref, b_ref, out_ref):
        core = ca_ref[...] * ga_ref[...] + cb_ref[...] * gb_ref[...]
        out_ref[...] = h2_ref[...] + _ln(t_ref[...] + core, g_ref[...], b_ref[...])

    row_spec = pl.BlockSpec((blk, _D), lambda i: (i, 0))
    col_spec = pl.BlockSpec((blk, 1), lambda i: (i, 0))
    vec_spec = pl.BlockSpec((1, _D), lambda i: (0, 0))
    return pl.pallas_call(
        body,
        grid=(nb,),
        in_specs=[row_spec, row_spec, col_spec, col_spec, row_spec, row_spec,
                  vec_spec, vec_spec],
        out_specs=row_spec,
        out_shape=jax.ShapeDtypeStruct((_T, _D), jnp.float32),
    )(ga, gb, ca, cb, t, h2, g, b)


def _logits(h, g, b, tok_emb):
    vb = 640
    nv = _V // vb

    def body(h_ref, g_ref, b_ref, emb_ref, out_ref, hn_ref):
        j = pl.program_id(0)

        @pl.when(j == 0)
        def _():
            hn_ref[...] = _ln(h_ref[...], g_ref[...], b_ref[...])

        out_ref[...] = lax.dot_general(hn_ref[...], emb_ref[...],
                                       (((1,), (1,)), ((), ())),
                                       preferred_element_type=jnp.float32)

    return pl.pallas_call(
        body,
        grid=(nv,),
        in_specs=[pl.BlockSpec((_T, _D), lambda j: (0, 0)),
                  pl.BlockSpec((1, _D), lambda j: (0, 0)),
                  pl.BlockSpec((1, _D), lambda j: (0, 0)),
                  pl.BlockSpec((vb, _D), lambda j: (j, 0))],
        out_specs=pl.BlockSpec((_T, vb), lambda j: (0, j)),
        out_shape=jax.ShapeDtypeStruct((_T, _V), jnp.float32),
        scratch_shapes=[pltpu.VMEM((_T, _D), jnp.float32)],
    )(h, g, b, tok_emb)


def kernel(x, tok_emb, pos_emb, ln_attn_g, ln_attn_b, wq, bq, wk, bk, wv, bv,
           wo, bo, ln_ff_g, ln_ff_b, moe_ln_g, moe_ln_b, wg, w1, b1, w2, b2,
           ln_head_g, ln_head_b):
    x_flat = x.reshape(_T)
    pad_row = (x_flat == 0).astype(jnp.float32).reshape(1, _T)
    zeros_ecap = jnp.zeros((_ECAP,), jnp.int32)
    r1 = lambda a: a.reshape(1, _D)

    h0 = _sc_gather(tok_emb, x_flat, 64)

    h = None
    for l in range(_L):
        if l == 0:
            q, k, v, hres = _make_lnqkv(True)(
                h0, pos_emb, r1(ln_attn_g[l]), r1(ln_attn_b[l]),
                wq[l], r1(bq[l]), wk[l], r1(bk[l]), wv[l], r1(bv[l]))
        else:
            q, k, v = _make_lnqkv(False)(
                h, r1(ln_attn_g[l]), r1(ln_attn_b[l]),
                wq[l], r1(bq[l]), wk[l], r1(bk[l]), wv[l], r1(bv[l]))
            hres = h
        hd = lambda a: a.reshape(_T, _H, _DH).transpose(1, 0, 2)
        o3 = _attn(hd(q), hd(k), hd(v), pad_row)
        o = o3.transpose(1, 0, 2).reshape(_T, _D)
        h2, t = _oproj(o, hres, wo[l], r1(bo[l]), r1(ln_ff_g[l]), r1(ln_ff_b[l]))
        sa, sb, ca, cb, va, vb = _router(t, wg[l])
        sa1, sb1 = sa.reshape(_T), sb.reshape(_T)
        eb = _sc_dispatch(t, sa1, sb1, va.reshape(_T), vb.reshape(_T), zeros_ecap)
        eo = _expert_ffn(eb.reshape(_E, _CAP, _D), w1[l], b1[l], w2[l],
                         b2[l]).reshape(_ECAP, _D)
        ga = _sc_gather(eo, sa1, 64)
        gb = _sc_gather(eo, sb1, 64)
        h = _combine(ga, gb, ca, cb, t, h2, r1(moe_ln_g[l]), r1(moe_ln_b[l]))

    lg = _logits(h, r1(ln_head_g), r1(ln_head_b), tok_emb)
    return lg.reshape(_B, _S, _V)


# speedup vs baseline: 1.1631x; 1.1011x over previous
import functools

import jax
import jax.numpy as jnp
from jax import lax
from jax.experimental import pallas as pl
from jax.experimental.pallas import tpu as pltpu
from jax.experimental.pallas import tpu_sc as plsc

_B, _S, _D, _H, _DH, _L, _E, _K, _V, _DFF = 1, 2048, 768, 12, 64, 2, 8, 2, 32000, 3072
_T = _B * _S
_CAP = 640
_ECAP = _E * _CAP

_NC, _NS = 2, 16
_NW = _NC * _NS

def _sc_mesh():
    return dict(
        mesh=plsc.VectorSubcoreMesh(
            core_axis_name="c", subcore_axis_name="s", num_cores=_NC),
        compiler_params=pltpu.CompilerParams(needs_layout_passes=False))


def _worker_id():
    return lax.axis_index("s") * _NC + lax.axis_index("c")


def _sc_gather(table, idx, chunk):
    n, d = idx.shape[0], table.shape[1]
    per_w = n // _NW
    assert per_w % chunk == 0 and chunk % 8 == 0

    @functools.partial(
        pl.kernel,
        out_type=jax.ShapeDtypeStruct((n, d), jnp.float32),
        scratch_types=[
            pltpu.VMEM((chunk,), jnp.int32),
            pltpu.VMEM((chunk, d), jnp.float32),
            pltpu.SemaphoreType.DMA,
        ],
        **_sc_mesh(),
    )
    def k(table_hbm, idx_hbm, out_hbm, idx_v, rows_v, sem):
        base0 = _worker_id() * per_w
        for c in range(per_w // chunk):
            base = base0 + c * chunk
            pltpu.sync_copy(idx_hbm.at[pl.ds(base, chunk)], idx_v)
            pltpu.async_copy(table_hbm.at[idx_v], rows_v, sem).wait()
            pltpu.sync_copy(rows_v, out_hbm.at[pl.ds(base, chunk)])

    return k(table, idx)


def _sc_dispatch(xt, slot_a, slot_b):
    per_w = _T // _NW

    @functools.partial(
        pl.kernel,
        out_type=jax.ShapeDtypeStruct((_ECAP + 8, _D), jnp.float32),
        scratch_types=[
            pltpu.VMEM((per_w,), jnp.int32),
            pltpu.VMEM((per_w,), jnp.int32),
            pltpu.VMEM((per_w, _D), jnp.float32),
            pltpu.SemaphoreType.DMA,
        ],
        **_sc_mesh(),
    )
    def k(xt_hbm, sa_hbm, sb_hbm, out_hbm, sa_v, sb_v, rows_v, sem):
        base = _worker_id() * per_w
        pltpu.sync_copy(sa_hbm.at[pl.ds(base, per_w)], sa_v)
        pltpu.sync_copy(sb_hbm.at[pl.ds(base, per_w)], sb_v)
        pltpu.sync_copy(xt_hbm.at[pl.ds(base, per_w)], rows_v)
        pltpu.async_copy(rows_v, out_hbm.at[sa_v], sem).wait()
        pltpu.async_copy(rows_v, out_hbm.at[sb_v], sem).wait()

    return k(xt, slot_a, slot_b)


def _ln(h, g, b):
    m = jnp.mean(h, axis=-1, keepdims=True)
    v = jnp.mean((h - m) ** 2, axis=-1, keepdims=True)
    return (h - m) / jnp.sqrt(v + 1e-5) * g + b


def _make_lnqkv(with_pos):
    nb = 8
    blk = _T // nb

    def body(*refs):
        if with_pos:
            (h_ref, pos_ref, g_ref, b_ref, wq_ref, bq_ref, wk_ref, bk_ref,
             wv_ref, bv_ref, q_ref, k_ref, v_ref, hres_ref) = refs
            hh = h_ref[...] + pos_ref[...]
            hres_ref[...] = hh
        else:
            (h_ref, g_ref, b_ref, wq_ref, bq_ref, wk_ref, bk_ref,
             wv_ref, bv_ref, q_ref, k_ref, v_ref) = refs
            hh = h_ref[...]
        a = _ln(hh, g_ref[...], b_ref[...])
        q_ref[...] = jnp.dot(a, wq_ref[...], preferred_element_type=jnp.float32) + bq_ref[...]
        k_ref[...] = jnp.dot(a, wk_ref[...], preferred_element_type=jnp.float32) + bk_ref[...]
        v_ref[...] = jnp.dot(a, wv_ref[...], preferred_element_type=jnp.float32) + bv_ref[...]

    row_spec = pl.BlockSpec((blk, _D), lambda i: (i, 0))
    full_spec = pl.BlockSpec((_D, _D), lambda i: (0, 0))
    vec_spec = pl.BlockSpec((1, _D), lambda i: (0, 0))
    n_out = 4 if with_pos else 3
    in_specs = [row_spec] * (2 if with_pos else 1) + [vec_spec, vec_spec] + \
        [full_spec, vec_spec] * 3
    out_specs = [row_spec] * n_out
    out_shape = [jax.ShapeDtypeStruct((_T, _D), jnp.float32)] * n_out
    return pl.pallas_call(body, grid=(nb,), in_specs=in_specs,
                          out_specs=out_specs, out_shape=out_shape)


def _attn(q, k, v, pad_row):
    qb = 512
    nq = _T // qb

    def body(q_ref, k_ref, v_ref, pad_ref, o_ref):
        qi = pl.program_id(1)
        qv = q_ref[...].reshape(qb, _DH)
        kv = k_ref[...].reshape(_T, _DH)
        s = lax.dot_general(qv, kv, (((1,), (1,)), ((), ())),
                            preferred_element_type=jnp.float32) / 8.0
        rows = qi * qb + lax.broadcasted_iota(jnp.int32, (qb, _T), 0)
        cols = lax.broadcasted_iota(jnp.int32, (qb, _T), 1)
        masked = (cols > rows) | (pad_ref[...] > 0.0)
        s = jnp.where(masked, -1e9, s)
        m = jnp.max(s, axis=1, keepdims=True)
        p = jnp.exp(s - m)
        l = jnp.sum(p, axis=1, keepdims=True)
        o = lax.dot_general(p, v_ref[...].reshape(_T, _DH),
                            (((1,), (0,)), ((), ())),
                            preferred_element_type=jnp.float32) / l
        o_ref[...] = o.reshape(1, qb, _DH)

    return pl.pallas_call(
        body,
        grid=(_H, nq),
        in_specs=[
            pl.BlockSpec((1, qb, _DH), lambda h, i: (h, i, 0)),
            pl.BlockSpec((1, _T, _DH), lambda h, i: (h, 0, 0)),
            pl.BlockSpec((1, _T, _DH), lambda h, i: (h, 0, 0)),
            pl.BlockSpec((1, _T), lambda h, i: (0, 0)),
        ],
        out_specs=pl.BlockSpec((1, qb, _DH), lambda h, i: (h, i, 0)),
        out_shape=jax.ShapeDtypeStruct((_H, _T, _DH), jnp.float32),
    )(q, k, v, pad_row)


def _oproj(o, h, wo, bo, g, b):
    nb = 8
    blk = _T // nb

    def body(o_ref, h_ref, wo_ref, bo_ref, g_ref, b_ref, h2_ref, t_ref):
        h2 = h_ref[...] + jnp.dot(o_ref[...], wo_ref[...],
                                  preferred_element_type=jnp.float32) + bo_ref[...]
        h2_ref[...] = h2
        t_ref[...] = _ln(h2, g_ref[...], b_ref[...])

    row_spec = pl.BlockSpec((blk, _D), lambda i: (i, 0))
    return pl.pallas_call(
        body,
        grid=(nb,),
        in_specs=[row_spec, row_spec,
                  pl.BlockSpec((_D, _D), lambda i: (0, 0)),
                  pl.BlockSpec((1, _D), lambda i: (0, 0)),
                  pl.BlockSpec((1, _D), lambda i: (0, 0)),
                  pl.BlockSpec((1, _D), lambda i: (0, 0))],
        out_specs=[row_spec, row_spec],
        out_shape=[jax.ShapeDtypeStruct((_T, _D), jnp.float32)] * 2,
    )(o, h, wo, bo, g, b)


def _router(xt, wg):
    cb = 256
    nb = _T // cb

    def body(xt_ref, wg_ref, sa_ref, sb_ref, ca_ref, cb_ref, va_ref, vb_ref):
        gl = jnp.dot(xt_ref[...], wg_ref[...], preferred_element_type=jnp.float32)
        iota_e = lax.broadcasted_iota(jnp.int32, (_T, _E), 1)
        m1 = jnp.max(gl, axis=1, keepdims=True)
        i1 = jnp.min(jnp.where(gl == m1, iota_e, _E), axis=1, keepdims=True)
        gl2 = jnp.where(iota_e == i1, -1e30, gl)
        m2 = jnp.max(gl2, axis=1, keepdims=True)
        i2 = jnp.min(jnp.where(gl2 == m2, iota_e, _E), axis=1, keepdims=True)
        e = jnp.exp(m2 - m1)
        g1 = 1.0 / (1.0 + e)
        g2 = e / (1.0 + e)
        oha = (iota_e == i1).astype(jnp.float32)
        ohb = (iota_e == i2).astype(jnp.float32)
        oh = oha + ohb
        tri = (lax.broadcasted_iota(jnp.int32, (cb, cb), 1)
               < lax.broadcasted_iota(jnp.int32, (cb, cb), 0)).astype(jnp.float32)
        run = jnp.zeros((1, _E), jnp.float32)
        cs = []
        for bi in range(nb):
            blk = oh[bi * cb:(bi + 1) * cb]
            cs.append(jnp.dot(tri, blk, preferred_element_type=jnp.float32) + run)
            run = run + jnp.sum(blk, axis=0, keepdims=True)
        c = jnp.concatenate(cs, axis=0)
        pie_a = jnp.sum(c * oha, axis=1, keepdims=True).astype(jnp.int32)
        pie_b = jnp.sum(c * ohb, axis=1, keepdims=True).astype(jnp.int32)
        va = pie_a < _CAP
        vb = pie_b < _CAP
        sa = i1 * _CAP + jnp.minimum(pie_a, _CAP - 1)
        sb = i2 * _CAP + jnp.minimum(pie_b, _CAP - 1)
        sa_ref[...] = sa
        sb_ref[...] = sb
        va_ref[...] = jnp.where(va, sa, _ECAP)
        vb_ref[...] = jnp.where(vb, sb, _ECAP)
        ca_ref[...] = g1 * va.astype(jnp.float32)
        cb_ref[...] = g2 * vb.astype(jnp.float32)

    col_i = jax.ShapeDtypeStruct((_T, 1), jnp.int32)
    col_f = jax.ShapeDtypeStruct((_T, 1), jnp.float32)
    return pl.pallas_call(
        body,
        in_specs=[pl.BlockSpec((_T, _D), lambda: (0, 0)),
                  pl.BlockSpec((_D, _E), lambda: (0, 0))],
        out_specs=[pl.BlockSpec((_T, 1), lambda: (0, 0))] * 6,
        out_shape=[col_i, col_i, col_f, col_f, col_i, col_i],
    )(xt, wg)


def _expert_ffn(eb, w1, b1, w2, b2):
    nj = _DFF // _D

    def body(eb_ref, w1_ref, b1_ref, w2_ref, b2_ref, out_ref):
        j = pl.program_id(1)
        ebv = eb_ref[...].reshape(_CAP, _D)
        hid = jnp.maximum(
            jnp.dot(ebv, w1_ref[...].reshape(_D, _D),
                    preferred_element_type=jnp.float32)
            + b1_ref[...].reshape(1, _D), 0.0)
        part = jnp.dot(hid, w2_ref[...].reshape(_D, _D),
                       preferred_element_type=jnp.float32)

        @pl.when(j == 0)
        def _():
            out_ref[...] = (part + b2_ref[...].reshape(1, _D))[None]

        @pl.when(j > 0)
        def _():
            out_ref[...] = out_ref[...] + part[None]

    return pl.pallas_call(
        body,
        grid=(_E, nj),
        in_specs=[
            pl.BlockSpec((1, _CAP, _D), lambda e, j: (e, 0, 0)),
            pl.BlockSpec((1, _D, _D), lambda e, j: (e, 0, j)),
            pl.BlockSpec((1, 1, _D), lambda e, j: (e, 0, j)),
            pl.BlockSpec((1, _D, _D), lambda e, j: (e, j, 0)),
            pl.BlockSpec((1, 1, _D), lambda e, j: (e, 0, 0)),
        ],
        out_specs=pl.BlockSpec((1, _CAP, _D), lambda e, j: (e, 0, 0)),
        out_shape=jax.ShapeDtypeStruct((_E, _CAP, _D), jnp.float32),
    )(eb, w1, b1.reshape(_E, 1, _DFF), w2, b2.reshape(_E, 1, _D))


def _combine(ga, gb, ca, cb, t, h2, g, b):
    nb = 8
    blk = _T // nb

    def body(ga_ref, gb_ref, ca_ref, cb_ref, t_ref, h2_ref, g_ref, b_ref, out_ref):
        core = ca_ref[...] * ga_ref[...] + cb_ref[...] * gb_ref[...]
        out_ref[...] = h2_ref[...] + _ln(t_ref[...] + core, g_ref[...], b_ref[...])

    row_spec = pl.BlockSpec((blk, _D), lambda i: (i, 0))
    col_spec = pl.BlockSpec((blk, 1), lambda i: (i, 0))
    vec_spec = pl.BlockSpec((1, _D), lambda i: (0, 0))
    return pl.pallas_call(
        body,
        grid=(nb,),
        in_specs=[row_spec, row_spec, col_spec, col_spec, row_spec, row_spec,
                  vec_spec, vec_spec],
        out_specs=row_spec,
        out_shape=jax.ShapeDtypeStruct((_T, _D), jnp.float32),
    )(ga, gb, ca, cb, t, h2, g, b)


def _logits(h, g, b, tok_emb):
    vb = 640
    nv = _V // vb

    def body(h_ref, g_ref, b_ref, emb_ref, out_ref, hn_ref):
        j = pl.program_id(0)

        @pl.when(j == 0)
        def _():
            hn_ref[...] = _ln(h_ref[...], g_ref[...], b_ref[...])

        out_ref[...] = lax.dot_general(hn_ref[...], emb_ref[...],
                                       (((1,), (1,)), ((), ())),
                                       preferred_element_type=jnp.float32)

    return pl.pallas_call(
        body,
        grid=(nv,),
        in_specs=[pl.BlockSpec((_T, _D), lambda j: (0, 0)),
                  pl.BlockSpec((1, _D), lambda j: (0, 0)),
                  pl.BlockSpec((1, _D), lambda j: (0, 0)),
                  pl.BlockSpec((vb, _D), lambda j: (j, 0))],
        out_specs=pl.BlockSpec((_T, vb), lambda j: (0, j)),
        out_shape=jax.ShapeDtypeStruct((_T, _V), jnp.float32),
        scratch_shapes=[pltpu.VMEM((_T, _D), jnp.float32)],
    )(h, g, b, tok_emb)


def kernel(x, tok_emb, pos_emb, ln_attn_g, ln_attn_b, wq, bq, wk, bk, wv, bv,
           wo, bo, ln_ff_g, ln_ff_b, moe_ln_g, moe_ln_b, wg, w1, b1, w2, b2,
           ln_head_g, ln_head_b):
    x_flat = x.reshape(_T)
    pad_row = (x_flat == 0).astype(jnp.float32).reshape(1, _T)
    r1 = lambda a: a.reshape(1, _D)

    h0 = _sc_gather(tok_emb, x_flat, 64)

    h = None
    for l in range(_L):
        if l == 0:
            q, k, v, hres = _make_lnqkv(True)(
                h0, pos_emb, r1(ln_attn_g[l]), r1(ln_attn_b[l]),
                wq[l], r1(bq[l]), wk[l], r1(bk[l]), wv[l], r1(bv[l]))
        else:
            q, k, v = _make_lnqkv(False)(
                h, r1(ln_attn_g[l]), r1(ln_attn_b[l]),
                wq[l], r1(bq[l]), wk[l], r1(bk[l]), wv[l], r1(bv[l]))
            hres = h
        hd = lambda a: a.reshape(_T, _H, _DH).transpose(1, 0, 2)
        o3 = _attn(hd(q), hd(k), hd(v), pad_row)
        o = o3.transpose(1, 0, 2).reshape(_T, _D)
        h2, t = _oproj(o, hres, wo[l], r1(bo[l]), r1(ln_ff_g[l]), r1(ln_ff_b[l]))
        sa, sb, ca, cb, sas, sbs = _router(t, wg[l])
        sa1, sb1 = sa.reshape(_T), sb.reshape(_T)
        eb = _sc_dispatch(t, sas.reshape(_T), sbs.reshape(_T))
        eo = _expert_ffn(eb[:_ECAP].reshape(_E, _CAP, _D), w1[l], b1[l], w2[l],
                         b2[l]).reshape(_ECAP, _D)
        ga = _sc_gather(eo, sa1, 64)
        gb = _sc_gather(eo, sb1, 64)
        h = _combine(ga, gb, ca, cb, t, h2, r1(moe_ln_g[l]), r1(moe_ln_b[l]))

    lg = _logits(h, r1(ln_head_g), r1(ln_head_b), tok_emb)
    return lg.reshape(_B, _S, _V)
